# baseline (speedup 1.0000x reference)
import jax
import jax.numpy as jnp
from jax.experimental import pallas as pl

N, E, D = 10000, 320000, 128


def _head_body(pooled_ref, wf1_ref, bf1_ref, wf2_ref, bf2_ref, out_ref):
    p = pooled_ref[...]
    h = jnp.maximum(
        jnp.dot(p, wf1_ref[...], preferred_element_type=jnp.float32)
        + bf1_ref[...],
        0.0,
    )
    o = jnp.dot(h, wf2_ref[...], preferred_element_type=jnp.float32) + bf2_ref[...]
    out_ref[...] = o[0:1]


def kernel(x, edge_index, W1, b1, g1, be1, W2, b2, g2, be2, Wf1, bf1, Wf2, bf2):
    src = edge_index[0]
    dst = edge_index[1]

    def gcn(h, W, b, g, be):
        h = h @ W
        deg_out = jnp.zeros((N,), jnp.float32).at[src].add(1.0)
        deg_in = jnp.zeros((N,), jnp.float32).at[dst].add(1.0)
        ns = jnp.where(deg_out > 0, deg_out ** -0.5, 0.0)
        nd = jnp.where(deg_in > 0, deg_in ** -0.5, 0.0)
        msg = h[src] * ns[src][:, None]
        agg = jnp.zeros_like(h).at[dst].add(msg)
        agg = (agg * nd[:, None] + b) * g + be
        return jax.nn.relu(agg)

    h = gcn(x, W1, b1, g1, be1)
    h = gcn(h, W2, b2, g2, be2)
    pooled = jnp.max(h, axis=0, keepdims=True)
    pooled8 = jnp.broadcast_to(pooled, (8, D))
    out = pl.pallas_call(
        _head_body,
        out_shape=jax.ShapeDtypeStruct((1, 256), jnp.float32),
    )(pooled8, Wf1, bf1.reshape(1, -1), Wf2, bf2.reshape(1, -1))
    return out


# speedup vs baseline: 7.6007x; 7.6007x over previous
import functools

import jax
import jax.numpy as jnp
from jax import lax
from jax.experimental import pallas as pl
from jax.experimental.pallas import tpu as pltpu
from jax.experimental.pallas import tpu_sc as plsc

N, E, D = 10000, 320000, 128
NP = 10240
NCORES, NSUB, NW = 2, 16, 32
C = 128
EPW = 10112
NCH = EPW // C
EPAD = NW * EPW - E
ACC_N = 10112
RT = ACC_N // NSUB
BN = 1024
GRID = NP // BN

_sc_mesh = plsc.VectorSubcoreMesh(core_axis_name="c", subcore_axis_name="s")
_sc_params = pltpu.CompilerParams(needs_layout_passes=False)


@functools.partial(
    pl.kernel,
    out_type=jax.ShapeDtypeStruct((2 * NW, NP // 128, 128), jnp.float32),
    mesh=_sc_mesh,
    compiler_params=_sc_params,
    scratch_types=[
        pltpu.VMEM((NCH, C), jnp.int32),
        pltpu.VMEM((NCH, C), jnp.int32),
        pltpu.VMEM((NP // 128, 128), jnp.float32),
        pltpu.VMEM((NP // 128, 128), jnp.float32),
    ],
)
def _deg_kernel(src_hbm, dst_hbm, out_hbm, srcb, dstb, dsrc, ddst):
    cid = lax.axis_index("c")
    sid = lax.axis_index("s")
    wid = cid * NSUB + sid
    zeros16 = jnp.zeros((16,), jnp.float32)
    ones16 = jnp.ones((16,), jnp.float32)

    def zbody(j, carry):
        for cc in range(8):
            dsrc[j, pl.ds(cc * 16, 16)] = zeros16
            ddst[j, pl.ds(cc * 16, 16)] = zeros16
        return carry

    lax.fori_loop(0, NP // 128, zbody, 0)

    pltpu.sync_copy(src_hbm.at[wid], srcb)
    pltpu.sync_copy(dst_hbm.at[wid], dstb)

    def sbody(j, carry):
        for k in range(8):
            idx = srcb[j, pl.ds(k * 16, 16)]
            r = jax.lax.shift_right_logical(idx, 7)
            c = jax.lax.bitwise_and(idx, 127)
            plsc.addupdate_scatter(dsrc, [r, c], ones16)
            idx2 = dstb[j, pl.ds(k * 16, 16)]
            r2 = jax.lax.shift_right_logical(idx2, 7)
            c2 = jax.lax.bitwise_and(idx2, 127)
            plsc.addupdate_scatter(ddst, [r2, c2], ones16)
        return carry

    lax.fori_loop(0, NCH, sbody, 0)

    pltpu.sync_copy(dsrc, out_hbm.at[wid])
    pltpu.sync_copy(ddst, out_hbm.at[NW + wid])


@functools.partial(
    pl.kernel,
    out_type=jax.ShapeDtypeStruct((2 * NP, D), jnp.float32),
    mesh=_sc_mesh,
    compiler_params=_sc_params,
    scratch_types=[
        pltpu.VMEM((NCH, C), jnp.int32),
        pltpu.VMEM((NCH, C), jnp.int32),
        pltpu.VMEM((C, D), jnp.float32),
        pltpu.VMEM_SHARED((ACC_N, D), jnp.float32),
        pltpu.SemaphoreType.DMA,
    ],
)
def _scat_kernel(h_hbm, src_hbm, dst_hbm, zrows_hbm, out_hbm,
                 srcb, dstb, rowsb, acc, gsem):
    cid = lax.axis_index("c")
    sid = lax.axis_index("s")
    wid = cid * NSUB + sid

    pltpu.sync_copy(zrows_hbm, acc.at[pl.ds(sid * RT, RT)])
    pltpu.sync_copy(src_hbm.at[wid], srcb)
    pltpu.sync_copy(dst_hbm.at[wid], dstb)
    plsc.subcore_barrier()

    def chunk(i, carry):
        pltpu.async_copy(h_hbm.at[srcb.at[i]], rowsb, gsem).wait()
        pltpu.sync_copy(rowsb, acc.at[dstb.at[i]], add=True)
        return carry

    lax.fori_loop(0, NCH, chunk, 0)

    plsc.subcore_barrier()
    pltpu.sync_copy(acc.at[pl.ds(sid * RT, RT)],
                    out_hbm.at[pl.ds(cid * NP + sid * RT, RT)])


def _norms_body(degp_ref, ns_ref, nd_ref):
    m = degp_ref[...]
    s = jnp.sum(m[0:NW], axis=0, keepdims=True)
    d = jnp.sum(m[NW:2 * NW], axis=0, keepdims=True)
    ns = jnp.where(s > 0, lax.rsqrt(s), 0.0)
    nd = jnp.where(d > 0, lax.rsqrt(d), 0.0)
    nsT = jnp.transpose(ns, (1, 0))
    ndT = jnp.transpose(nd, (1, 0))
    ns_ref[...] = jnp.broadcast_to(nsT, (BN, D))
    nd_ref[...] = jnp.broadcast_to(ndT, (BN, D))


_norms_call = pl.pallas_call(
    _norms_body,
    grid=(GRID,),
    in_specs=[pl.BlockSpec((2 * NW, BN), lambda i: (0, i))],
    out_specs=[
        pl.BlockSpec((BN, D), lambda i: (i, 0)),
        pl.BlockSpec((BN, D), lambda i: (i, 0)),
    ],
    out_shape=[
        jax.ShapeDtypeStruct((NP, D), jnp.float32),
        jax.ShapeDtypeStruct((NP, D), jnp.float32),
    ],
)


def _mm1_body(x_ref, w_ref, ns_ref, o_ref):
    h = jnp.dot(x_ref[...], w_ref[...], preferred_element_type=jnp.float32)
    o_ref[...] = h * ns_ref[...]


_mm1_call = pl.pallas_call(
    _mm1_body,
    grid=(GRID,),
    in_specs=[
        pl.BlockSpec((BN, D), lambda i: (i, 0)),
        pl.BlockSpec((D, D), lambda i: (0, 0)),
        pl.BlockSpec((BN, D), lambda i: (i, 0)),
    ],
    out_specs=pl.BlockSpec((BN, D), lambda i: (i, 0)),
    out_shape=jax.ShapeDtypeStruct((NP, D), jnp.float32),
)


def _mid_body(p0_ref, p1_ref, nd_ref, ns_ref, sc_ref, sh_ref, w_ref, o_ref):
    p = p0_ref[...] + p1_ref[...]
    y = jnp.maximum(p * nd_ref[...] * sc_ref[...] + sh_ref[...], 0.0)
    h = jnp.dot(y, w_ref[...], preferred_element_type=jnp.float32)
    o_ref[...] = h * ns_ref[...]


_mid_call = pl.pallas_call(
    _mid_body,
    grid=(GRID,),
    in_specs=[
        pl.BlockSpec((BN, D), lambda i: (i, 0)),
        pl.BlockSpec((BN, D), lambda i: (i + GRID, 0)),
        pl.BlockSpec((BN, D), lambda i: (i, 0)),
        pl.BlockSpec((BN, D), lambda i: (i, 0)),
        pl.BlockSpec((1, D), lambda i: (0, 0)),
        pl.BlockSpec((1, D), lambda i: (0, 0)),
        pl.BlockSpec((D, D), lambda i: (0, 0)),
    ],
    out_specs=pl.BlockSpec((BN, D), lambda i: (i, 0)),
    out_shape=jax.ShapeDtypeStruct((NP, D), jnp.float32),
)


def _fin_body(parts_ref, nd_ref, sc_ref, sh_ref,
              wf1_ref, bf1_ref, wf2_ref, bf2_ref, o_ref):
    p = parts_ref[0:NP] + parts_ref[NP:2 * NP]
    y = jnp.maximum(p * nd_ref[...] * sc_ref[...] + sh_ref[...], 0.0)
    rows = lax.broadcasted_iota(jnp.int32, (NP, 1), 0)
    y = jnp.where(rows < N, y, -jnp.inf)
    pooled = jnp.max(y, axis=0, keepdims=True)
    p8 = jnp.broadcast_to(pooled, (8, D))
    h1 = jnp.maximum(
        jnp.dot(p8, wf1_ref[...], preferred_element_type=jnp.float32)
        + bf1_ref[...], 0.0)
    o = jnp.dot(h1, wf2_ref[...], preferred_element_type=jnp.float32) \
        + bf2_ref[...]
    o_ref[...] = o[0:1]


_fin_call = pl.pallas_call(
    _fin_body,
    out_shape=jax.ShapeDtypeStruct((1, 256), jnp.float32),
)


def kernel(x, edge_index, W1, b1, g1, be1, W2, b2, g2, be2, Wf1, bf1, Wf2, bf2):
    src = edge_index[0]
    dst = edge_index[1]
    src3 = jnp.concatenate(
        [src, jnp.full((EPAD,), N, jnp.int32)]).reshape(NW, NCH, C)
    dst3 = jnp.concatenate(
        [dst, jnp.full((EPAD,), N + 16, jnp.int32)]).reshape(NW, NCH, C)
    zrows = jnp.zeros((RT, D), jnp.float32)

    degp = _deg_kernel(src3, dst3).reshape(2 * NW, NP)
    ns_b, nd_b = _norms_call(degp)

    xp = jnp.pad(x, ((0, NP - N), (0, 0)))
    h1s = _mm1_call(xp, W1, ns_b)
    parts1 = _scat_kernel(h1s, src3, dst3, zrows)

    sc1 = g1.reshape(1, D)
    sh1 = (b1 * g1 + be1).reshape(1, D)
    h2s = _mid_call(parts1, parts1, nd_b, ns_b, sc1, sh1, W2)
    parts2 = _scat_kernel(h2s, src3, dst3, zrows)

    sc2 = g2.reshape(1, D)
    sh2 = (b2 * g2 + be2).reshape(1, D)
    out = _fin_call(parts2, nd_b, sc2, sh2,
                    Wf1, bf1.reshape(1, -1), Wf2, bf2.reshape(1, -1))
    return out
